# baseline (speedup 1.0000x reference)
import functools

import jax
import jax.numpy as jnp
from jax.experimental import pallas as pl
from jax.experimental.pallas import tpu as pltpu

_VMEM_LIMIT = 64 * 1024 * 1024
_N = 8


def _canv(H, W):
    return (H + 2) * (W + 2) + 8


def _col_mask(M, Wp, W):
    r = jax.lax.broadcasted_iota(jnp.int32, (M, 1), 0)
    w = jax.lax.rem(r + 1, Wp)
    return jnp.logical_and(w >= 1, w <= W)


def _taps9(canv, M, Wp):
    cols = []
    for di in range(3):
        for dj in range(3):
            o = di * Wp + dj
            cols.append(canv[o:o + M, :])
    return jnp.concatenate(cols, axis=1)


def _stem_kernel(p_ref, w_ref, b_ref, o_ref):
    p = p_ref[0]
    y = jnp.dot(p, w_ref[...], preferred_element_type=jnp.float32)
    y = jnp.maximum(y + b_ref[...], 0.0).astype(jnp.bfloat16)
    v = y.reshape(112, 2, 56, 64)
    ye = v[:, 0]
    yo = v[:, 1]
    yo_l = jnp.pad(yo, ((0, 0), (1, 0), (0, 0)))[:, :56, :]
    wmax = jnp.maximum(jnp.maximum(yo_l, ye), yo)
    hp = jnp.pad(wmax, ((1, 1), (0, 0), (0, 0)))
    h4 = hp.reshape(57, 2, 56, 64)
    m = jnp.maximum(jnp.maximum(h4[0:56, 0], h4[0:56, 1]), h4[1:57, 0])
    o_ref[0] = jnp.pad(m, ((0, 0), (1, 1), (0, 0)))


def _ublock_kernel(x_ref, w1_ref, b1_ref, w2_ref, b2_ref, o_ref, *,
                   H, W, C):
    Wp = W + 2
    M = H * Wp
    CANV = _canv(H, W)
    x = x_ref[0]
    mask = _col_mask(M, Wp, W)
    p1 = _taps9(x, M, Wp)
    y = jnp.dot(p1, w1_ref[...], preferred_element_type=jnp.float32)
    y = jnp.where(mask, jnp.maximum(y + b1_ref[...], 0.0), 0.0)
    yc = jnp.pad(y.astype(jnp.bfloat16),
                 ((Wp + 1, CANV - Wp - 1 - M), (0, 0)))
    p2 = _taps9(yc, M, Wp)
    ident = x[Wp + 1:Wp + 1 + M, :].astype(jnp.float32)
    z = jnp.dot(p2, w2_ref[...], preferred_element_type=jnp.float32)
    z = jnp.where(mask, jnp.maximum(z + b2_ref[...] + ident, 0.0), 0.0)
    o_ref[0] = jnp.pad(z.astype(jnp.bfloat16),
                       ((Wp + 1, CANV - Wp - 1 - M), (0, 0)))


def _tblock_kernel(p1_ref, xd_ref, w1_ref, b1_ref, w2_ref, b2_ref,
                   wd_ref, bd_ref, o_ref, *, H2, W2, Cout):
    Wp = W2 + 2
    M = H2 * Wp
    CANV = _canv(H2, W2)
    mask = _col_mask(M, Wp, W2)
    y = jnp.dot(p1_ref[0], w1_ref[...], preferred_element_type=jnp.float32)
    y = jnp.where(mask, jnp.maximum(y + b1_ref[...], 0.0), 0.0)
    yc = jnp.pad(y.astype(jnp.bfloat16),
                 ((Wp + 1, CANV - Wp - 1 - M), (0, 0)))
    p2 = _taps9(yc, M, Wp)
    idacc = jnp.dot(xd_ref[0], wd_ref[...],
                    preferred_element_type=jnp.float32) + bd_ref[...]
    ident = idacc.astype(jnp.bfloat16).astype(jnp.float32)
    z = jnp.dot(p2, w2_ref[...], preferred_element_type=jnp.float32)
    z = jnp.where(mask, jnp.maximum(z + b2_ref[...] + ident, 0.0), 0.0)
    o_ref[0] = jnp.pad(z.astype(jnp.bfloat16),
                       ((Wp + 1, CANV - Wp - 1 - M), (0, 0)))


def _head_body(x_ref, w1_ref, b1_ref, w2_ref, b2_ref, w3_ref, b3_ref, o_ref):
    x = x_ref[...].astype(jnp.float32)
    h = jnp.sum(x, axis=1) * (1.0 / 49.0)
    h = jnp.dot(h.astype(jnp.bfloat16), w1_ref[...],
                preferred_element_type=jnp.float32) + b1_ref[...]
    h = jnp.dot(h.astype(jnp.bfloat16), w2_ref[...],
                preferred_element_type=jnp.float32) + b2_ref[...]
    h = jnp.dot(h.astype(jnp.bfloat16), w3_ref[...],
                preferred_element_type=jnp.float32) + b3_ref[...]
    o_ref[0] = h


def _full(shape):
    n = len(shape)
    return pl.BlockSpec(shape, lambda i, _n=n: (0,) * _n)


def _cparams():
    return pltpu.CompilerParams(
        dimension_semantics=("parallel",),
        vmem_limit_bytes=_VMEM_LIMIT)


def _stem(patches, w, b):
    return pl.pallas_call(
        _stem_kernel,
        out_shape=jax.ShapeDtypeStruct((_N, 56, 58, 64), jnp.bfloat16),
        grid=(_N,),
        in_specs=[
            pl.BlockSpec((1, 12544, 147), lambda n: (n, 0, 0)),
            _full((147, 64)),
            _full((1, 64)),
        ],
        out_specs=pl.BlockSpec((1, 56, 58, 64), lambda n: (n, 0, 0, 0)),
        compiler_params=_cparams(),
    )(patches, w, b)


def _ublock(x, w1, b1, w2, b2, H, W, C):
    CANV = _canv(H, W)
    kern = functools.partial(_ublock_kernel, H=H, W=W, C=C)
    return pl.pallas_call(
        kern,
        out_shape=jax.ShapeDtypeStruct((_N, CANV, C), jnp.bfloat16),
        grid=(_N,),
        in_specs=[
            pl.BlockSpec((1, CANV, C), lambda n: (n, 0, 0)),
            _full((9 * C, C)), _full((1, C)),
            _full((9 * C, C)), _full((1, C)),
        ],
        out_specs=pl.BlockSpec((1, CANV, C), lambda n: (n, 0, 0)),
        compiler_params=_cparams(),
    )(x, w1, b1, w2, b2)


def _tblock(p1, xd, w1, b1, w2, b2, wd, bd, H2, W2, Cin, Cout):
    CANV = _canv(H2, W2)
    M = H2 * (W2 + 2)
    kern = functools.partial(_tblock_kernel, H2=H2, W2=W2, Cout=Cout)
    return pl.pallas_call(
        kern,
        out_shape=jax.ShapeDtypeStruct((_N, CANV, Cout), jnp.bfloat16),
        grid=(_N,),
        in_specs=[
            pl.BlockSpec((1, M, 9 * Cin), lambda n: (n, 0, 0)),
            pl.BlockSpec((1, M, Cin), lambda n: (n, 0, 0)),
            _full((9 * Cin, Cout)), _full((1, Cout)),
            _full((9 * Cout, Cout)), _full((1, Cout)),
            _full((Cin, Cout)), _full((1, Cout)),
        ],
        out_specs=pl.BlockSpec((1, CANV, Cout), lambda n: (n, 0, 0)),
        compiler_params=_cparams(),
    )(p1, xd, w1, b1, w2, b2, wd, bd)


def _head(x, w1, b1, w2, b2, w3, b3):
    CANV4 = _canv(7, 7)
    out = pl.pallas_call(
        _head_body,
        out_shape=jax.ShapeDtypeStruct((2, 4, 1), jnp.float32),
        grid=(2,),
        in_specs=[
            pl.BlockSpec((4, CANV4, 512), lambda i: (i, 0, 0)),
            _full((512, 256)), _full((1, 256)),
            _full((256, 32)), _full((1, 32)),
            _full((32, 1)), _full((1, 1)),
        ],
        out_specs=pl.BlockSpec((1, 4, 1), lambda i: (i, 0, 0)),
        compiler_params=_cparams(),
    )(x, w1, b1, w2, b2, w3, b3)
    return out.reshape(-1)


def _stem_patches(x):
    xt = jnp.transpose(x, (0, 2, 3, 1)).astype(jnp.bfloat16)
    xp = jnp.pad(xt, ((0, 0), (3, 3), (3, 3), (0, 0)))
    cols = []
    for i in range(7):
        for j in range(7):
            cols.append(xp[:, i:i + 223:2, j:j + 223:2, :])
    p = jnp.stack(cols, axis=3)
    p = p.reshape(_N, 112, 56, 2, 147).transpose(0, 1, 3, 2, 4)
    return p.reshape(_N, 12544, 147)


def _canvas_from_4d(core, H, W, C):
    CANV = _canv(H, W)
    Wp = W + 2
    flat = core.reshape(_N, H * Wp, C)
    return jnp.pad(flat, ((0, 0), (Wp, CANV - Wp - H * Wp), (0, 0)))


def _trans_prep(a, H, W, C):
    Wp = W + 2
    H2, W2 = H // 2, W // 2
    x4 = a[:, Wp:Wp + H * Wp, :].reshape(_N, H, Wp, C)
    xp4 = jnp.pad(x4, ((0, 0), (1, 1), (0, 0), (0, 0)))
    cols = []
    for di in range(3):
        for dj in range(3):
            cols.append(xp4[:, di:di + H - 1:2, dj:dj + W - 1:2, :])
    p = jnp.stack(cols, axis=3)
    p = p.reshape(_N, H2, W2, 9 * C)
    p = jnp.pad(p, ((0, 0), (0, 0), (1, 1), (0, 0)))
    p1 = _shift1(p.reshape(_N, H2 * (W2 + 2), 9 * C))
    xd = xp4[:, 1:H + 1:2, 1:W + 1:2, :]
    xd = jnp.pad(xd, ((0, 0), (0, 0), (1, 1), (0, 0)))
    xd = _shift1(xd.reshape(_N, H2 * (W2 + 2), C))
    return p1, xd


def _shift1(a):
    return jnp.pad(a[:, 1:, :], ((0, 0), (0, 1), (0, 0)))


def _wflat(w):
    kh, kw, cin, cout = w.shape
    return w.reshape(kh * kw * cin, cout)


def kernel(p00, p01, p02, p03, p04, p05, p06, p07, p08, p09, p10, p11, p12, p13, p14, p15, p16, p17, p18, p19, p20, p21, p22, p23, p24, p25, p26, p27, p28, p29, p30, p31, p32, p33, p34, p35, p36, p37, p38, p39, p40, p41, p42, p43, p44, p45, p46, p47, p48, p49, p50, p51, p52, p53, p54, p55, p56, p57, p58, p59, p60, p61, p62, p63, p64, p65, p66, p67, p68, p69, p70, p71, p72, p73, p74, p75, p76, p77, x):
    leaves = [
        p00, p01, p02, p03, p04, p05, p06, p07, p08, p09,
        p10, p11, p12, p13, p14, p15, p16, p17, p18, p19,
        p20, p21, p22, p23, p24, p25, p26, p27, p28, p29,
        p30, p31, p32, p33, p34, p35, p36, p37, p38, p39,
        p40, p41, p42, p43, p44, p45, p46, p47, p48, p49,
        p50, p51, p52, p53, p54, p55, p56, p57, p58, p59,
        p60, p61, p62, p63, p64, p65, p66, p67, p68, p69,
        p70, p71, p72, p73, p74, p75, p76, p77,
    ]
    it = iter(leaves)

    def _pair():
        return (next(it), next(it))

    stem_w, stem_b = _pair()
    layers_cfg = [(64, 3), (128, 4), (256, 6), (512, 3)]
    in_ch = 64
    layers = []
    for planes, blocks in layers_cfg:
        layer = []
        for bi in range(blocks):
            blk = {"conv1": _pair(), "conv2": _pair()}
            if bi == 0 and in_ch != planes:
                blk["down"] = _pair()
            layer.append(blk)
            in_ch = planes
        layers.append(layer)
    fc = _pair()
    lin1 = _pair()
    lin2 = _pair()

    pooled = _stem(_stem_patches(x), stem_w.reshape(147, 64), stem_b)
    a = _canvas_from_4d(pooled, 56, 56, 64)

    dims = [(56, 64), (28, 128), (14, 256), (7, 512)]
    for li, layer in enumerate(layers):
        H, C = dims[li]
        for blk in layer:
            w1, b1 = blk["conv1"]
            w2, b2 = blk["conv2"]
            if "down" in blk:
                wd, bd = blk["down"]
                Hin, Cin = dims[li - 1]
                p1, xd = _trans_prep(a, Hin, Hin, Cin)
                a = _tblock(p1, xd, _wflat(w1), b1, _wflat(w2), b2,
                            wd.reshape(Cin, C), bd, H, H, Cin, C)
            else:
                a = _ublock(a, _wflat(w1), b1, _wflat(w2), b2, H, H, C)

    return _head(a, fc[0], fc[1], lin1[0], lin1[1], lin2[0], lin2[1])


# speedup vs baseline: 6.7464x; 6.7464x over previous
import functools

import jax
import jax.numpy as jnp
from jax.experimental import pallas as pl
from jax.experimental.pallas import tpu as pltpu

_VMEM_LIMIT = 64 * 1024 * 1024
_N = 8


def _canv(H, W):
    return (H + 2) * (W + 2) + 8


def _col_mask(M, Wp, W):
    r = jax.lax.broadcasted_iota(jnp.int32, (M, 1), 0)
    w = jax.lax.rem(r + 1, Wp)
    return jnp.logical_and(w >= 1, w <= W)


def _taps9(canv, M, Wp):
    cols = []
    for di in range(3):
        for dj in range(3):
            o = di * Wp + dj
            cols.append(canv[o:o + M, :])
    return jnp.concatenate(cols, axis=1)


def _taps9_into(ps_ref, canv, M, Wp):
    C = canv.shape[-1]
    for di in range(3):
        for dj in range(3):
            o = di * Wp + dj
            t = (di * 3 + dj) * C
            ps_ref[:, t:t + C] = canv[o:o + M, :]


def _conv9(canv, M, Wp, w_ref):
    C = canv.shape[-1]
    acc = None
    for di in range(3):
        for dj in range(3):
            o = di * Wp + dj
            t = (di * 3 + dj) * C
            d = jnp.dot(canv[o:o + M, :], w_ref[t:t + C, :],
                        preferred_element_type=jnp.float32)
            acc = d if acc is None else acc + d
    return acc


def _tree_dot(p, w_ref, k_lo, k_hi):
    cs = [jnp.dot(p[:, k0:min(k0 + 128, k_hi)],
                  w_ref[k0:min(k0 + 128, k_hi), :],
                  preferred_element_type=jnp.float32)
          for k0 in range(k_lo, k_hi, 128)]
    pairs = [cs[i] + cs[i + 1] for i in range(0, len(cs) - 1, 2)]
    rem = cs[-1] if len(cs) % 2 else None
    acc = pairs[0] if pairs else rem
    for pr in pairs[1:]:
        acc = acc + pr
    if pairs and rem is not None:
        acc = acc + rem
    return acc


def _dot_ref_order(p, w_ref, K):
    if K % 1152 == 0 and K > 1152:
        acc = None
        for k0 in range(0, K, 1152):
            d = jnp.dot(p[:, k0:k0 + 1152], w_ref[k0:k0 + 1152, :],
                        preferred_element_type=jnp.float32)
            acc = d if acc is None else acc + d
        return acc
    return jnp.dot(p, w_ref[...], preferred_element_type=jnp.float32)


def _stem_kernel(pe_ref, po_ref, w_ref, b_ref, o_ref):
    CANV = _canv(56, 56)
    ye = jnp.dot(pe_ref[0], w_ref[...], preferred_element_type=jnp.float32)
    ye = jnp.maximum(ye + b_ref[...], 0.0).astype(jnp.bfloat16)
    yo = jnp.dot(po_ref[0], w_ref[...], preferred_element_type=jnp.float32)
    yo = jnp.maximum(yo + b_ref[...], 0.0).astype(jnp.bfloat16)
    ye = ye.reshape(112, 56, 64)
    yo = yo.reshape(112, 56, 64)
    yo_l = jnp.pad(yo, ((0, 0), (1, 0), (0, 0)))[:, :56, :]
    wmax = jnp.maximum(jnp.maximum(yo_l, ye), yo)
    hp = jnp.pad(wmax, ((1, 1), (0, 0), (0, 0)))
    h4 = hp.reshape(57, 2, 56, 64)
    m = jnp.maximum(jnp.maximum(h4[0:56, 0], h4[0:56, 1]), h4[1:57, 0])
    mc = jnp.pad(m, ((0, 0), (1, 1), (0, 0)))
    flat = mc.reshape(56 * 58, 64)
    o_ref[0] = jnp.pad(flat, ((58, CANV - 58 - 56 * 58), (0, 0)))


def _ublock_kernel(x_ref, w1_ref, b1_ref, w2_ref, b2_ref, o_ref, ps_ref, *,
                   H, W, C):
    Wp = W + 2
    M = H * Wp
    CANV = _canv(H, W)
    x = x_ref[0]
    mask = _col_mask(M, Wp, W)
    if C == 64:
        y = _conv9(x, M, Wp, w1_ref)
    else:
        _taps9_into(ps_ref, x, M, Wp)
        y = _dot_ref_order(ps_ref[...], w1_ref, 9 * C)
    y = jnp.maximum(y + b1_ref[...], 0.0).astype(jnp.bfloat16)
    y = jnp.where(mask, y, jnp.bfloat16(0))
    yc = jnp.pad(y, ((Wp + 1, CANV - Wp - 1 - M), (0, 0)))
    ident = x[Wp + 1:Wp + 1 + M, :].astype(jnp.float32)
    if C == 64:
        z = _conv9(yc, M, Wp, w2_ref)
    else:
        _taps9_into(ps_ref, yc, M, Wp)
        z = _dot_ref_order(ps_ref[...], w2_ref, 9 * C)
    z = jnp.maximum(z + b2_ref[...] + ident, 0.0).astype(jnp.bfloat16)
    z = jnp.where(mask, z, jnp.bfloat16(0))
    o_ref[0] = jnp.pad(z, ((Wp + 1, CANV - Wp - 1 - M), (0, 0)))


def _tblock_kernel(ae_ref, ao_ref, w1_ref, b1_ref, w2_ref, b2_ref,
                   wd_ref, bd_ref, o_ref, p1_ref, p2_ref, *,
                   H, W, Cin, Cout):
    Hp = H + 2
    Wg = (W + 2) // 2
    H2, W2 = H // 2, W // 2
    Wp = W2 + 2
    M = H2 * Wp
    CANV = _canv(H2, W2)
    ae = ae_ref[0].reshape(Hp // 2, 2, Wg, Cin)
    ao = ao_ref[0].reshape(Hp // 2, 2, Wg, Cin)

    def s2tap(di, dj):
        pln = ao if dj % 2 else ae
        return pln[di // 2:di // 2 + H2, di % 2,
                   dj // 2:dj // 2 + W2, :].reshape(H2 * W2, Cin)

    for di in range(3):
        for dj in range(3):
            t = (di * 3 + dj) * Cin
            p1_ref[:, t:t + Cin] = s2tap(di, dj)
    y = _dot_ref_order(p1_ref[...], w1_ref, 9 * Cin)
    y = jnp.maximum(y + b1_ref[...], 0.0).astype(jnp.bfloat16)
    yg = jnp.pad(y.reshape(H2, W2, Cout),
                 ((0, 0), (1, 1), (0, 0))).reshape(M, Cout)
    yc = jnp.pad(yg, ((Wp, CANV - Wp - M), (0, 0)))
    _taps9_into(p2_ref, yc, M, Wp)
    xd = ao[0:H2, 1, 0:W2, :].reshape(H2 * W2, Cin)
    idacc = jnp.dot(xd, wd_ref[...],
                    preferred_element_type=jnp.float32) + bd_ref[...]
    idg = jnp.pad(idacc.astype(jnp.bfloat16).reshape(H2, W2, Cout),
                  ((0, 0), (1, 1), (0, 0))).reshape(M, Cout)
    ident = jnp.pad(idg[1:], ((0, 1), (0, 0))).astype(jnp.float32)
    mask = _col_mask(M, Wp, W2)
    z = _dot_ref_order(p2_ref[...], w2_ref, 9 * Cout)
    z = jnp.where(mask, jnp.maximum(z + b2_ref[...] + ident, 0.0), 0.0)
    o_ref[0] = jnp.pad(z.astype(jnp.bfloat16),
                       ((Wp + 1, CANV - Wp - 1 - M), (0, 0)))


def _head_body(x_ref, w1_ref, b1_ref, w2_ref, b2_ref, w3_ref, b3_ref, o_ref):
    x = x_ref[...].astype(jnp.float32)
    h = jnp.sum(x, axis=1) / 49.0
    h = jnp.dot(h.astype(jnp.bfloat16), w1_ref[...],
                preferred_element_type=jnp.float32) + b1_ref[...]
    h = jnp.dot(h.astype(jnp.bfloat16), w2_ref[...],
                preferred_element_type=jnp.float32) + b2_ref[...]
    h = jnp.dot(h.astype(jnp.bfloat16), w3_ref[...],
                preferred_element_type=jnp.float32) + b3_ref[...]
    o_ref[0] = h


def _full(shape):
    n = len(shape)
    return pl.BlockSpec(shape, lambda i, _n=n: (0,) * _n)


def _cparams():
    return pltpu.CompilerParams(
        dimension_semantics=("parallel",),
        vmem_limit_bytes=_VMEM_LIMIT)


def _stem(pe, po, w, b):
    CANV = _canv(56, 56)
    return pl.pallas_call(
        _stem_kernel,
        out_shape=jax.ShapeDtypeStruct((_N, CANV, 64), jnp.bfloat16),
        grid=(_N,),
        in_specs=[
            pl.BlockSpec((1, 6272, 147), lambda n: (n, 0, 0)),
            pl.BlockSpec((1, 6272, 147), lambda n: (n, 0, 0)),
            _full((147, 64)),
            _full((1, 64)),
        ],
        out_specs=pl.BlockSpec((1, CANV, 64), lambda n: (n, 0, 0)),
        compiler_params=_cparams(),
    )(pe, po, w, b)


def _ublock(x, w1, b1, w2, b2, H, W, C):
    CANV = _canv(H, W)
    kern = functools.partial(_ublock_kernel, H=H, W=W, C=C)
    return pl.pallas_call(
        kern,
        out_shape=jax.ShapeDtypeStruct((_N, CANV, C), jnp.bfloat16),
        grid=(_N,),
        in_specs=[
            pl.BlockSpec((1, CANV, C), lambda n: (n, 0, 0)),
            _full((9 * C, C)), _full((1, C)),
            _full((9 * C, C)), _full((1, C)),
        ],
        out_specs=pl.BlockSpec((1, CANV, C), lambda n: (n, 0, 0)),
        scratch_shapes=[pltpu.VMEM((H * (W + 2), 9 * C), jnp.bfloat16)],
        compiler_params=_cparams(),
    )(x, w1, b1, w2, b2)


def _tblock(a, w1, b1, w2, b2, wd, bd, H, W, Cin, Cout):
    Hp = H + 2
    Wp_in = W + 2
    Wg = Wp_in // 2
    H2, W2 = H // 2, W // 2
    CANV = _canv(H2, W2)
    core = a[:, :Hp * Wp_in, :].reshape(_N, Hp, Wp_in, Cin)
    ae = core[:, :, 0::2, :]
    ao = core[:, :, 1::2, :]
    kern = functools.partial(_tblock_kernel, H=H, W=W, Cin=Cin, Cout=Cout)
    return pl.pallas_call(
        kern,
        out_shape=jax.ShapeDtypeStruct((_N, CANV, Cout), jnp.bfloat16),
        grid=(_N,),
        in_specs=[
            pl.BlockSpec((1, Hp, Wg, Cin), lambda n: (n, 0, 0, 0)),
            pl.BlockSpec((1, Hp, Wg, Cin), lambda n: (n, 0, 0, 0)),
            _full((9 * Cin, Cout)), _full((1, Cout)),
            _full((9 * Cout, Cout)), _full((1, Cout)),
            _full((Cin, Cout)), _full((1, Cout)),
        ],
        out_specs=pl.BlockSpec((1, CANV, Cout), lambda n: (n, 0, 0)),
        scratch_shapes=[
            pltpu.VMEM((H2 * W2, 9 * Cin), jnp.bfloat16),
            pltpu.VMEM((H2 * (W2 + 2), 9 * Cout), jnp.bfloat16),
        ],
        compiler_params=_cparams(),
    )(ae, ao, w1, b1, w2, b2, wd, bd)


def _head(x, w1, b1, w2, b2, w3, b3):
    CANV4 = _canv(7, 7)
    out = pl.pallas_call(
        _head_body,
        out_shape=jax.ShapeDtypeStruct((2, 4, 1), jnp.float32),
        grid=(2,),
        in_specs=[
            pl.BlockSpec((4, CANV4, 512), lambda i: (i, 0, 0)),
            _full((512, 256)), _full((1, 256)),
            _full((256, 32)), _full((1, 32)),
            _full((32, 1)), _full((1, 1)),
        ],
        out_specs=pl.BlockSpec((1, 4, 1), lambda i: (i, 0, 0)),
        compiler_params=_cparams(),
    )(x, w1, b1, w2, b2, w3, b3)
    return out.reshape(-1)


def _stem_patches(x):
    xt = jnp.transpose(x, (0, 2, 3, 1)).astype(jnp.bfloat16)
    xp = jnp.pad(xt, ((0, 0), (3, 3), (3, 3), (0, 0)))
    ce, co = [], []
    for i in range(7):
        for j in range(7):
            ce.append(xp[:, i:i + 223:2, j:j + 221:4, :])
            co.append(xp[:, i:i + 223:2, j + 2:j + 223:4, :])
    pe = jnp.stack(ce, axis=3).reshape(_N, 6272, 147)
    po = jnp.stack(co, axis=3).reshape(_N, 6272, 147)
    return pe, po


def _wflat(w):
    kh, kw, cin, cout = w.shape
    return w.reshape(kh * kw * cin, cout)


def kernel(p00, p01, p02, p03, p04, p05, p06, p07, p08, p09, p10, p11, p12, p13, p14, p15, p16, p17, p18, p19, p20, p21, p22, p23, p24, p25, p26, p27, p28, p29, p30, p31, p32, p33, p34, p35, p36, p37, p38, p39, p40, p41, p42, p43, p44, p45, p46, p47, p48, p49, p50, p51, p52, p53, p54, p55, p56, p57, p58, p59, p60, p61, p62, p63, p64, p65, p66, p67, p68, p69, p70, p71, p72, p73, p74, p75, p76, p77, x):
    leaves = [
        p00, p01, p02, p03, p04, p05, p06, p07, p08, p09,
        p10, p11, p12, p13, p14, p15, p16, p17, p18, p19,
        p20, p21, p22, p23, p24, p25, p26, p27, p28, p29,
        p30, p31, p32, p33, p34, p35, p36, p37, p38, p39,
        p40, p41, p42, p43, p44, p45, p46, p47, p48, p49,
        p50, p51, p52, p53, p54, p55, p56, p57, p58, p59,
        p60, p61, p62, p63, p64, p65, p66, p67, p68, p69,
        p70, p71, p72, p73, p74, p75, p76, p77,
    ]
    it = iter(leaves)

    def _pair():
        return (next(it), next(it))

    stem_w, stem_b = _pair()
    layers_cfg = [(64, 3), (128, 4), (256, 6), (512, 3)]
    in_ch = 64
    layers = []
    for planes, blocks in layers_cfg:
        layer = []
        for bi in range(blocks):
            blk = {"conv1": _pair(), "conv2": _pair()}
            if bi == 0 and in_ch != planes:
                blk["down"] = _pair()
            layer.append(blk)
            in_ch = planes
        layers.append(layer)
    fc = _pair()
    lin1 = _pair()
    lin2 = _pair()

    pe, po = _stem_patches(x)
    a = _stem(pe, po, stem_w.reshape(147, 64), stem_b)

    dims = [(56, 64), (28, 128), (14, 256), (7, 512)]
    for li, layer in enumerate(layers):
        H, C = dims[li]
        for blk in layer:
            w1, b1 = blk["conv1"]
            w2, b2 = blk["conv2"]
            if "down" in blk:
                wd, bd = blk["down"]
                Hin, Cin = dims[li - 1]
                a = _tblock(a, _wflat(w1), b1, _wflat(w2), b2,
                            wd.reshape(Cin, C), bd, Hin, Hin, Cin, C)
            else:
                a = _ublock(a, _wflat(w1), b1, _wflat(w2), b2, H, H, C)

    return _head(a, fc[0], fc[1], lin1[0], lin1[1], lin2[0], lin2[1])
